# Initial kernel scaffold; baseline (speedup 1.0000x reference)
import functools
import math

import jax
import jax.numpy as jnp
from jax import lax
from jax.experimental import pallas as pl
from jax.experimental.pallas import tpu as pltpu
from jax.experimental.pallas import tpu_sc as plsc

BS = 4
NUM_MOL = 256
N = BS * NUM_MOL
E = 32768
C = 64
WM = 32
FEAT = 64

NW = 32
EPW = E // NW
LANES = 16

F32 = jnp.float32
_LOG_MAXLEN = 2.0 * math.log(2056.0) / 64.0


def _sc_agg_body(xs_hbm, ys_hbm, zs_hbm, row_hbm, col_hbm, out_hbm,
                 xs_v, ys_v, zs_v, row_v, col_v, acc_v):
    wid = lax.axis_index("s") * 2 + lax.axis_index("c")
    base = wid * EPW
    pltpu.sync_copy(xs_hbm, xs_v)
    pltpu.sync_copy(ys_hbm, ys_v)
    pltpu.sync_copy(zs_hbm, zs_v)
    pltpu.sync_copy(row_hbm.at[pl.ds(base, EPW)], row_v)
    pltpu.sync_copy(col_hbm.at[pl.ds(base, EPW)], col_v)

    def zero_body(i, carry):
        acc_v[pl.ds(i * LANES, LANES)] = jnp.zeros((LANES,), F32)
        return carry

    lax.fori_loop(0, (LANES * N) // LANES, zero_body, 0)

    lane_off = lax.iota(jnp.int32, LANES) * N

    def body(i, carry):
        r = row_v[pl.ds(i * LANES, LANES)]
        c = col_v[pl.ds(i * LANES, LANES)]
        dx = plsc.load_gather(xs_v, [r]) - plsc.load_gather(xs_v, [c])
        dy = plsc.load_gather(ys_v, [r]) - plsc.load_gather(ys_v, [c])
        dz = plsc.load_gather(zs_v, [r]) - plsc.load_gather(zs_v, [c])
        rad = dx * dx + dy * dy + dz * dz
        plsc.addupdate_scatter(acc_v, [lane_off + r], rad)
        return carry

    lax.fori_loop(0, EPW // LANES, body, 0)
    pltpu.sync_copy(acc_v, out_hbm.at[wid])


def _sc_agg(xs, ys, zs, row, col):
    mesh = plsc.VectorSubcoreMesh(core_axis_name="c", subcore_axis_name="s")
    k = functools.partial(
        pl.kernel,
        out_type=jax.ShapeDtypeStruct((NW, LANES * N), F32),
        mesh=mesh,
        scratch_types=[
            pltpu.VMEM((N,), F32),
            pltpu.VMEM((N,), F32),
            pltpu.VMEM((N,), F32),
            pltpu.VMEM((EPW,), jnp.int32),
            pltpu.VMEM((EPW,), jnp.int32),
            pltpu.VMEM((LANES * N,), F32),
        ],
    )(_sc_agg_body)
    return k(xs, ys, zs, row, col)


def _k2_body(part_ref, ch_ref, at_ref, aatW, aatb, linW, linb, spW, spb,
             ne_ref, f1_ref):
    m_i = lax.broadcasted_iota(F32, (NUM_MOL, 32), 0)
    k_i = lax.broadcasted_iota(F32, (NUM_MOL, 32), 1)
    denom = jnp.exp(k_i * _LOG_MAXLEN)
    ang = m_i * math.pi / denom
    pe = jnp.concatenate([jnp.sin(ang), jnp.cos(ang)], axis=1)
    pe4 = jnp.concatenate([pe, pe, pe, pe], axis=0)
    ce = pe4 * ch_ref[...]
    ate = jnp.dot(at_ref[...], aatW[...], preferred_element_type=F32) + aatb[...]
    ne = jnp.dot(jnp.concatenate([ce, ate], axis=1), linW[...],
                 preferred_element_type=F32) + linb[...]
    ne_ref[...] = ne
    agg = jnp.sum(part_ref[...], axis=1, keepdims=True)
    pin = jnp.concatenate([ne, agg], axis=1)
    f1_ref[...] = jnp.dot(pin, spW[...], preferred_element_type=F32) + spb[...]


def _k2_call(part_t, ch2, at2, node_p, sp_p):
    return pl.pallas_call(
        _k2_body,
        out_shape=[jax.ShapeDtypeStruct((N, 3), F32),
                   jax.ShapeDtypeStruct((N, FEAT), F32)],
    )(part_t, ch2, at2,
      node_p["aatype"]["W"].T, node_p["aatype"]["b"].reshape(1, -1),
      node_p["lin"]["W"].T, node_p["lin"]["b"].reshape(1, -1),
      sp_p["W"].T, sp_p["b"].reshape(1, -1))


_K3_ROWS = NUM_MOL * NUM_MOL
_K3_BLK = 8192


def _k3_body(rW, rb, out_ref):
    blk = pl.program_id(0)
    r_loc = lax.broadcasted_iota(jnp.int32, (_K3_BLK, 32), 0) + blk * _K3_BLK
    rel = (r_loc // NUM_MOL - r_loc % NUM_MOL).astype(F32)
    k_i = lax.broadcasted_iota(F32, (_K3_BLK, 32), 1)
    denom = jnp.exp(k_i * _LOG_MAXLEN)
    ang = rel * math.pi / denom
    emb = jnp.concatenate([jnp.sin(ang), jnp.cos(ang)], axis=1)
    out_ref[...] = jnp.dot(emb, rW[...], preferred_element_type=F32) + rb[...]


def _k3_call(relpos_p):
    return pl.pallas_call(
        _k3_body,
        grid=(_K3_ROWS // _K3_BLK,),
        in_specs=[pl.BlockSpec((FEAT, FEAT), lambda i: (0, 0)),
                  pl.BlockSpec((1, FEAT), lambda i: (0, 0))],
        out_specs=pl.BlockSpec((_K3_BLK, FEAT), lambda i: (i, 0)),
        out_shape=jax.ShapeDtypeStruct((_K3_ROWS, FEAT), F32),
    )(relpos_p["W"].T, relpos_p["b"].reshape(1, -1))


_IC = 32
_NIC = NUM_MOL // _IC


def _k4_body(fi_ref, f1_ref, rp_ref, w1, b1, w2, b2, w3, b3, lng, lnb,
             wf, bf, out_ref):
    ic = pl.program_id(1)
    rows = _IC * NUM_MOL
    f1b = f1_ref[0]
    fi = fi_ref[0]
    fi_rep = jnp.broadcast_to(fi[:, None, :], (_IC, NUM_MOL, FEAT))
    fi_rep = fi_rep.reshape(rows, FEAT)
    fj_t = jnp.broadcast_to(f1b[None, :, :], (_IC, NUM_MOL, FEAT))
    fj_t = fj_t.reshape(rows, FEAT)
    x = jnp.concatenate([fi_rep, fj_t, rp_ref[...]], axis=1)
    h = jnp.dot(x, w1[...], preferred_element_type=F32) + b1[...]
    h = jnp.maximum(h, 0.0)
    h = jnp.dot(h, w2[...], preferred_element_type=F32) + b2[...]
    h = jnp.maximum(h, 0.0)
    h = jnp.dot(h, w3[...], preferred_element_type=F32) + b3[...]
    mu = jnp.mean(h, axis=1, keepdims=True)
    var = jnp.mean((h - mu) * (h - mu), axis=1, keepdims=True)
    h = (h - mu) / jnp.sqrt(var + 1e-5) * lng[...] + lnb[...]
    ef = jnp.dot(h, wf[...], preferred_element_type=F32) + bf[...]
    acc = jnp.sum(ef.reshape(_IC, NUM_MOL * 3), axis=0)
    acc = acc.reshape(1, NUM_MOL, 3)

    @pl.when(ic == 0)
    def _():
        out_ref[...] = acc

    @pl.when(ic != 0)
    def _():
        out_ref[...] = out_ref[...] + acc


def _k4_call(f1_3d, rp, edge_p):
    full = lambda shape: pl.BlockSpec(shape, lambda b, i: (0, 0))
    return pl.pallas_call(
        _k4_body,
        grid=(BS, _NIC),
        in_specs=[
            pl.BlockSpec((1, _IC, FEAT), lambda b, i: (b, i, 0)),
            pl.BlockSpec((1, NUM_MOL, FEAT), lambda b, i: (b, 0, 0)),
            pl.BlockSpec((_IC * NUM_MOL, FEAT), lambda b, i: (i, 0)),
            full((3 * FEAT, FEAT)), full((1, FEAT)),
            full((FEAT, FEAT)), full((1, FEAT)),
            full((FEAT, FEAT)), full((1, FEAT)),
            full((1, FEAT)), full((1, FEAT)),
            full((FEAT, 3)), full((1, 3)),
        ],
        out_specs=pl.BlockSpec((1, NUM_MOL, 3), lambda b, i: (b, 0, 0)),
        out_shape=jax.ShapeDtypeStruct((BS, NUM_MOL, 3), F32),
    )(f1_3d, f1_3d, rp,
      edge_p["e1"]["W"].T, edge_p["e1"]["b"].reshape(1, -1),
      edge_p["e2"]["W"].T, edge_p["e2"]["b"].reshape(1, -1),
      edge_p["e3"]["W"].T, edge_p["e3"]["b"].reshape(1, -1),
      edge_p["ln"]["g"].reshape(1, -1), edge_p["ln"]["b"].reshape(1, -1),
      edge_p["final"]["W"].T, edge_p["final"]["b"].reshape(1, -1))


def _pack_conv(p):
    w = p["W"]
    wst = jnp.transpose(w, (2, 3, 1, 0))
    wst = wst.reshape(3, 3 * w.shape[1], w.shape[0])
    bias = jnp.tile(p["b"].reshape(1, -1), (1, 3))
    return [wst, bias]


def _prep_cbn(p):
    return _pack_conv(p["conv"]) + [p["bn"]["g"].reshape(1, -1),
                                    p["bn"]["bta"].reshape(1, -1)]


class _Take:
    def __init__(self, refs):
        self.refs = list(refs)
        self.i = 0

    def __call__(self):
        r = self.refs[self.i]
        self.i += 1
        return r


def _shift_rows(t, dh, m0, m255):
    if dh == 1:
        return t
    z = jnp.zeros((1, t.shape[1]), F32)
    if dh == 0:
        return jnp.concatenate([z, t[:-1, :]], axis=0) * m0
    return jnp.concatenate([t[1:, :], z], axis=0) * m255


def _conv_u(take, x, cin, m0, m255):
    wref = take()
    bias = take()[...]
    outs = []
    for wo in range(3):
        acc = None
        for dh in range(3):
            w_dh = wref[dh]
            if wo == 0:
                t = jnp.dot(x[:, :2 * cin], w_dh[cin:, :],
                            preferred_element_type=F32)
            elif wo == 1:
                t = jnp.dot(x, w_dh, preferred_element_type=F32)
            else:
                t = jnp.dot(x[:, cin:], w_dh[:2 * cin, :],
                            preferred_element_type=F32)
            t = _shift_rows(t, dh, m0, m255)
            acc = t if acc is None else acc + t
        outs.append(acc)
    return jnp.concatenate(outs, axis=1) + bias


def _bn_lrelu(take, y, cout):
    g = take()[...]
    bta = take()[...]
    n = 3.0 * N
    s = jnp.sum(y, axis=0, keepdims=True)
    s2 = jnp.sum(y * y, axis=0, keepdims=True)
    mean = jnp.sum(s.reshape(3, cout), axis=0, keepdims=True) / n
    ex2 = jnp.sum(s2.reshape(3, cout), axis=0, keepdims=True) / n
    var = ex2 - mean * mean
    scale = g / jnp.sqrt(var + 1e-5)
    shift = bta - mean * scale
    scale3 = jnp.concatenate([scale, scale, scale], axis=1)
    shift3 = jnp.concatenate([shift, shift, shift], axis=1)
    y = y * scale3 + shift3
    return jnp.where(y >= 0, y, 0.01 * y)


def _cbn(take, x, cin, cout, m0, m255):
    return _bn_lrelu(take, _conv_u(take, x, cin, m0, m255), cout)


def _ccat(parts):
    cols = []
    for w in range(3):
        for p in parts:
            ci = p.shape[1] // 3
            cols.append(p[:, w * ci:(w + 1) * ci])
    return jnp.concatenate(cols, axis=1)


def _dense_u(take, x, cin, cout, m0, m255):
    t = 4 * cout
    out = _cbn(take, x, cin, t, m0, m255)
    return _cbn(take, _ccat([x, out]), cin + t, cout, m0, m255)


def _k5_body(*refs):
    pos_ref, ne_ref, ef_ref, m_ref = refs[:4]
    out_ref = refs[-1]
    take = _Take(refs[4:-1])

    h_idx = lax.broadcasted_iota(jnp.int32, (N, 1), 0) % NUM_MOL
    m0 = jnp.where(h_idx == 0, 0.0, 1.0)
    m255 = jnp.where(h_idx == NUM_MOL - 1, 0.0, 1.0)

    m2 = m_ref[...]
    x = _conv_u(take, pos_ref[...], 1, m0, m255)
    x = _conv_u(take, _ccat([x, ne_ref[...], ef_ref[...]]), C + 2, m0, m255)
    d1 = _dense_u(take, _ccat([x, m2]), C + WM, C, m0, m255)
    d2 = _dense_u(take, _ccat([x, d1, m2]), 2 * C + WM, C, m0, m255)
    d3 = _dense_u(take, _ccat([x, d1, d2, m2]), 3 * C + WM, C, m0, m255)
    a1 = _dense_u(take, d3, C, 2 * C, m0, m255)
    a2 = _dense_u(take, a1, 2 * C, 3 * C, m0, m255)
    a3 = _dense_u(take, a2, 3 * C, C, m0, m255)
    t0 = _cbn(take, _ccat([a3, m2]), C + WM, C, m0, m255)
    tt = _cbn(take, t0, C, WM, m0, m255)
    f0 = _cbn(take, a3, C, C, m0, m255)
    ff = _cbn(take, f0, C, WM, m0, m255)
    fparts = []
    for w in range(3):
        s = ff[:, w * WM:(w + 1) * WM]
        s = s - jnp.max(s, axis=1, keepdims=True)
        e = jnp.exp(s)
        fparts.append(e / jnp.sum(e, axis=1, keepdims=True))
    fsm = jnp.concatenate(fparts, axis=1)
    out_ref[...] = _conv_u(take, tt * fsm, WM, m0, m255)


def _k5_weights(params):
    ws = []
    ws += _pack_conv(params["first"])
    ws += _pack_conv(params["second"])
    for name in ("dl1", "dl2", "dl3", "da1", "da2", "da3"):
        ws += _prep_cbn(params[name]["c1"])
        ws += _prep_cbn(params[name]["c2"])
    ws += _prep_cbn(params["third"][0])
    ws += _prep_cbn(params["third"][1])
    ws += _prep_cbn(params["forth"][0])
    ws += _prep_cbn(params["forth"][1])
    ws += _pack_conv(params["final"])
    return ws


def _k5_call(pos2, ne2, ef2, m2, params):
    ws = _k5_weights(params)
    return pl.pallas_call(
        _k5_body,
        out_shape=jax.ShapeDtypeStruct((N, 3), F32),
    )(pos2, ne2, ef2, m2, *ws)


def kernel(position, message, charges, atom_types, edge_index, params):
    pos = position.astype(F32)
    xs, ys, zs = pos[:, 0], pos[:, 1], pos[:, 2]
    row = edge_index[0].astype(jnp.int32)
    col = edge_index[1].astype(jnp.int32)

    partials = _sc_agg(xs, ys, zs, row, col)
    part_t = partials.reshape(NW * LANES, N).T

    ch2 = charges.reshape(N, 1).astype(F32)
    at2 = atom_types.reshape(N, 5).astype(F32)
    ne2, f1 = _k2_call(part_t, ch2, at2, params["node"], params["edge"]["s_p"])

    rp = _k3_call(params["edge"]["relpos"])
    ef = _k4_call(f1.reshape(BS, NUM_MOL, FEAT), rp, params["edge"])
    ef2 = ef.reshape(N, 3)

    m2 = jnp.tile(message.transpose(0, 2, 1).reshape(N, WM), (1, 3))
    y = _k5_call(pos, ne2, ef2, m2, params)
    return y.reshape(BS, NUM_MOL, 3)[:, None]


# speedup vs baseline: 1.6724x; 1.6724x over previous
import functools
import math

import jax
import jax.numpy as jnp
from jax import lax
from jax.experimental import pallas as pl
from jax.experimental.pallas import tpu as pltpu
from jax.experimental.pallas import tpu_sc as plsc

BS = 4
NUM_MOL = 256
N = BS * NUM_MOL
E = 32768
C = 64
WM = 32
FEAT = 64

NW = 32
EPW = E // NW
LANES = 16

F32 = jnp.float32
_LOG_MAXLEN = 2.0 * math.log(2056.0) / 64.0


def _sc_agg_body(xs_hbm, ys_hbm, zs_hbm, row_hbm, col_hbm, out_hbm,
                 xs_v, ys_v, zs_v, row_v, col_v, acc_v):
    wid = lax.axis_index("s") * 2 + lax.axis_index("c")
    base = wid * EPW
    pltpu.sync_copy(xs_hbm, xs_v)
    pltpu.sync_copy(ys_hbm, ys_v)
    pltpu.sync_copy(zs_hbm, zs_v)
    pltpu.sync_copy(row_hbm.at[pl.ds(base, EPW)], row_v)
    pltpu.sync_copy(col_hbm.at[pl.ds(base, EPW)], col_v)

    def zero_body(i, carry):
        acc_v[pl.ds(i * LANES, LANES)] = jnp.zeros((LANES,), F32)
        return carry

    lax.fori_loop(0, (LANES * N) // LANES, zero_body, 0)

    lane_off = lax.iota(jnp.int32, LANES) * N

    def body(i, carry):
        r = row_v[pl.ds(i * LANES, LANES)]
        c = col_v[pl.ds(i * LANES, LANES)]
        dx = plsc.load_gather(xs_v, [r]) - plsc.load_gather(xs_v, [c])
        dy = plsc.load_gather(ys_v, [r]) - plsc.load_gather(ys_v, [c])
        dz = plsc.load_gather(zs_v, [r]) - plsc.load_gather(zs_v, [c])
        rad = dx * dx + dy * dy + dz * dz
        plsc.addupdate_scatter(acc_v, [lane_off + r], rad)
        return carry

    lax.fori_loop(0, EPW // LANES, body, 0)
    pltpu.sync_copy(acc_v, out_hbm.at[wid])


def _sc_agg(xs, ys, zs, row, col):
    mesh = plsc.VectorSubcoreMesh(core_axis_name="c", subcore_axis_name="s")
    k = functools.partial(
        pl.kernel,
        out_type=jax.ShapeDtypeStruct((NW, LANES * N), F32),
        mesh=mesh,
        compiler_params=pltpu.CompilerParams(needs_layout_passes=False),
        scratch_types=[
            pltpu.VMEM((N,), F32),
            pltpu.VMEM((N,), F32),
            pltpu.VMEM((N,), F32),
            pltpu.VMEM((EPW,), jnp.int32),
            pltpu.VMEM((EPW,), jnp.int32),
            pltpu.VMEM((LANES * N,), F32),
        ],
    )(_sc_agg_body)
    return k(xs, ys, zs, row, col)


def _k2_body(part_ref, ch_ref, at_ref, aatW, aatb, linW, linb, spW, spb,
             ne_ref, f1_ref):
    m_i = lax.broadcasted_iota(jnp.int32, (NUM_MOL, 32), 0).astype(F32)
    k_i = lax.broadcasted_iota(jnp.int32, (NUM_MOL, 32), 1).astype(F32)
    denom = jnp.exp(k_i * _LOG_MAXLEN)
    ang = m_i * math.pi / denom
    pe = jnp.concatenate([jnp.sin(ang), jnp.cos(ang)], axis=1)
    pe4 = jnp.concatenate([pe, pe, pe, pe], axis=0)
    ce = pe4 * ch_ref[...]
    ate = jnp.dot(at_ref[...], aatW[...], preferred_element_type=F32) + aatb[...]
    ne = jnp.dot(jnp.concatenate([ce, ate], axis=1), linW[...],
                 preferred_element_type=F32) + linb[...]
    ne_ref[...] = ne
    agg = jnp.sum(part_ref[...], axis=1, keepdims=True)
    pin = jnp.concatenate([ne, agg], axis=1)
    f1_ref[...] = jnp.dot(pin, spW[...], preferred_element_type=F32) + spb[...]


def _k2_call(part_t, ch2, at2, node_p, sp_p):
    return pl.pallas_call(
        _k2_body,
        out_shape=[jax.ShapeDtypeStruct((N, 3), F32),
                   jax.ShapeDtypeStruct((N, FEAT), F32)],
    )(part_t, ch2, at2,
      node_p["aatype"]["W"].T, node_p["aatype"]["b"].reshape(1, -1),
      node_p["lin"]["W"].T, node_p["lin"]["b"].reshape(1, -1),
      sp_p["W"].T, sp_p["b"].reshape(1, -1))


_K3_ROWS = NUM_MOL * NUM_MOL
_K3_BLK = 8192


def _k3_body(rW, rb, out_ref):
    blk = pl.program_id(0)
    r_loc = lax.broadcasted_iota(jnp.int32, (_K3_BLK, 32), 0) + blk * _K3_BLK
    rel = (r_loc // NUM_MOL - r_loc % NUM_MOL).astype(F32)
    k_i = lax.broadcasted_iota(jnp.int32, (_K3_BLK, 32), 1).astype(F32)
    denom = jnp.exp(k_i * _LOG_MAXLEN)
    ang = rel * math.pi / denom
    emb = jnp.concatenate([jnp.sin(ang), jnp.cos(ang)], axis=1)
    out_ref[...] = jnp.dot(emb, rW[...], preferred_element_type=F32) + rb[...]


def _k3_call(relpos_p):
    return pl.pallas_call(
        _k3_body,
        grid=(_K3_ROWS // _K3_BLK,),
        in_specs=[pl.BlockSpec((FEAT, FEAT), lambda i: (0, 0)),
                  pl.BlockSpec((1, FEAT), lambda i: (0, 0))],
        out_specs=pl.BlockSpec((_K3_BLK, FEAT), lambda i: (i, 0)),
        out_shape=jax.ShapeDtypeStruct((_K3_ROWS, FEAT), F32),
    )(relpos_p["W"].T, relpos_p["b"].reshape(1, -1))


_IC = 32
_NIC = NUM_MOL // _IC


def _k4_body(fi_ref, f1_ref, rp_ref, w1, b1, w2, b2, w3, b3, lng, lnb,
             wf, bf, out_ref):
    ic = pl.program_id(1)
    rows = _IC * NUM_MOL
    f1b = f1_ref[0]
    fi = fi_ref[0]
    fi_rep = jnp.broadcast_to(fi[:, None, :], (_IC, NUM_MOL, FEAT))
    fi_rep = fi_rep.reshape(rows, FEAT)
    fj_t = jnp.broadcast_to(f1b[None, :, :], (_IC, NUM_MOL, FEAT))
    fj_t = fj_t.reshape(rows, FEAT)
    x = jnp.concatenate([fi_rep, fj_t, rp_ref[...]], axis=1)
    h = jnp.dot(x, w1[...], preferred_element_type=F32) + b1[...]
    h = jnp.maximum(h, 0.0)
    h = jnp.dot(h, w2[...], preferred_element_type=F32) + b2[...]
    h = jnp.maximum(h, 0.0)
    h = jnp.dot(h, w3[...], preferred_element_type=F32) + b3[...]
    mu = jnp.mean(h, axis=1, keepdims=True)
    var = jnp.mean((h - mu) * (h - mu), axis=1, keepdims=True)
    h = (h - mu) / jnp.sqrt(var + 1e-5) * lng[...] + lnb[...]
    ef = jnp.dot(h, wf[...], preferred_element_type=F32) + bf[...]
    acc = jnp.sum(ef.reshape(_IC, NUM_MOL, 3), axis=0)
    acc = acc.reshape(1, NUM_MOL, 3)

    @pl.when(ic == 0)
    def _():
        out_ref[...] = acc

    @pl.when(ic != 0)
    def _():
        out_ref[...] = out_ref[...] + acc


def _k4_call(f1_3d, rp, edge_p):
    full = lambda shape: pl.BlockSpec(shape, lambda b, i: (0, 0))
    return pl.pallas_call(
        _k4_body,
        grid=(BS, _NIC),
        in_specs=[
            pl.BlockSpec((1, _IC, FEAT), lambda b, i: (b, i, 0)),
            pl.BlockSpec((1, NUM_MOL, FEAT), lambda b, i: (b, 0, 0)),
            pl.BlockSpec((_IC * NUM_MOL, FEAT), lambda b, i: (i, 0)),
            full((3 * FEAT, FEAT)), full((1, FEAT)),
            full((FEAT, FEAT)), full((1, FEAT)),
            full((FEAT, FEAT)), full((1, FEAT)),
            full((1, FEAT)), full((1, FEAT)),
            full((FEAT, 3)), full((1, 3)),
        ],
        out_specs=pl.BlockSpec((1, NUM_MOL, 3), lambda b, i: (b, 0, 0)),
        out_shape=jax.ShapeDtypeStruct((BS, NUM_MOL, 3), F32),
    )(f1_3d, f1_3d, rp,
      edge_p["e1"]["W"].T, edge_p["e1"]["b"].reshape(1, -1),
      edge_p["e2"]["W"].T, edge_p["e2"]["b"].reshape(1, -1),
      edge_p["e3"]["W"].T, edge_p["e3"]["b"].reshape(1, -1),
      edge_p["ln"]["g"].reshape(1, -1), edge_p["ln"]["b"].reshape(1, -1),
      edge_p["final"]["W"].T, edge_p["final"]["b"].reshape(1, -1))


def _pack_conv(p):
    w = p["W"]
    wst = jnp.transpose(w, (2, 3, 1, 0))
    wst = wst.reshape(3, 3 * w.shape[1], w.shape[0])
    bias = jnp.tile(p["b"].reshape(1, -1), (1, 3))
    return [wst, bias]


def _prep_cbn(p):
    return _pack_conv(p["conv"]) + [p["bn"]["g"].reshape(1, -1),
                                    p["bn"]["bta"].reshape(1, -1)]


class _Take:
    def __init__(self, refs):
        self.refs = list(refs)
        self.i = 0

    def __call__(self):
        r = self.refs[self.i]
        self.i += 1
        return r


def _shift_rows(t, dh, m0, m255):
    if dh == 1:
        return t
    z = jnp.zeros((1, t.shape[1]), F32)
    if dh == 0:
        return jnp.concatenate([z, t[:-1, :]], axis=0) * m0
    return jnp.concatenate([t[1:, :], z], axis=0) * m255


def _conv_u(take, x, cin, m0, m255):
    wref = take()
    bias = take()[...]
    outs = []
    for wo in range(3):
        acc = None
        for dh in range(3):
            w_dh = wref[dh]
            if wo == 0:
                t = jnp.dot(x[:, :2 * cin], w_dh[cin:, :],
                            preferred_element_type=F32)
            elif wo == 1:
                t = jnp.dot(x, w_dh, preferred_element_type=F32)
            else:
                t = jnp.dot(x[:, cin:], w_dh[:2 * cin, :],
                            preferred_element_type=F32)
            t = _shift_rows(t, dh, m0, m255)
            acc = t if acc is None else acc + t
        outs.append(acc)
    return jnp.concatenate(outs, axis=1) + bias


def _bn_lrelu(take, y, cout):
    g = take()[...]
    bta = take()[...]
    n = 3.0 * N
    s = jnp.sum(y, axis=0, keepdims=True)
    s2 = jnp.sum(y * y, axis=0, keepdims=True)
    mean = (s[:, :cout] + s[:, cout:2 * cout] + s[:, 2 * cout:]) / n
    ex2 = (s2[:, :cout] + s2[:, cout:2 * cout] + s2[:, 2 * cout:]) / n
    var = ex2 - mean * mean
    scale = g / jnp.sqrt(var + 1e-5)
    shift = bta - mean * scale
    scale3 = jnp.concatenate([scale, scale, scale], axis=1)
    shift3 = jnp.concatenate([shift, shift, shift], axis=1)
    y = y * scale3 + shift3
    return jnp.where(y >= 0, y, 0.01 * y)


def _cbn(take, x, cin, cout, m0, m255):
    return _bn_lrelu(take, _conv_u(take, x, cin, m0, m255), cout)


def _ccat(parts):
    cols = []
    for w in range(3):
        for p in parts:
            ci = p.shape[1] // 3
            cols.append(p[:, w * ci:(w + 1) * ci])
    return jnp.concatenate(cols, axis=1)


def _dense_u(take, x, cin, cout, m0, m255):
    t = 4 * cout
    out = _cbn(take, x, cin, t, m0, m255)
    return _cbn(take, _ccat([x, out]), cin + t, cout, m0, m255)


def _k5_body(*refs):
    pos_ref, ne_ref, ef_ref, m_ref = refs[:4]
    out_ref = refs[-1]
    take = _Take(refs[4:-1])

    h_idx = lax.broadcasted_iota(jnp.int32, (N, 1), 0) % NUM_MOL
    m0 = jnp.where(h_idx == 0, 0.0, 1.0)
    m255 = jnp.where(h_idx == NUM_MOL - 1, 0.0, 1.0)

    m2 = m_ref[...]
    x = _conv_u(take, pos_ref[...], 1, m0, m255)
    x = _conv_u(take, _ccat([x, ne_ref[...], ef_ref[...]]), C + 2, m0, m255)
    d1 = _dense_u(take, _ccat([x, m2]), C + WM, C, m0, m255)
    d2 = _dense_u(take, _ccat([x, d1, m2]), 2 * C + WM, C, m0, m255)
    d3 = _dense_u(take, _ccat([x, d1, d2, m2]), 3 * C + WM, C, m0, m255)
    a1 = _dense_u(take, d3, C, 2 * C, m0, m255)
    a2 = _dense_u(take, a1, 2 * C, 3 * C, m0, m255)
    a3 = _dense_u(take, a2, 3 * C, C, m0, m255)
    t0 = _cbn(take, _ccat([a3, m2]), C + WM, C, m0, m255)
    tt = _cbn(take, t0, C, WM, m0, m255)
    f0 = _cbn(take, a3, C, C, m0, m255)
    ff = _cbn(take, f0, C, WM, m0, m255)
    fparts = []
    for w in range(3):
        s = ff[:, w * WM:(w + 1) * WM]
        s = s - jnp.max(s, axis=1, keepdims=True)
        e = jnp.exp(s)
        fparts.append(e / jnp.sum(e, axis=1, keepdims=True))
    fsm = jnp.concatenate(fparts, axis=1)
    out_ref[...] = _conv_u(take, tt * fsm, WM, m0, m255)


def _k5_weights(params):
    ws = []
    ws += _pack_conv(params["first"])
    ws += _pack_conv(params["second"])
    for name in ("dl1", "dl2", "dl3", "da1", "da2", "da3"):
        ws += _prep_cbn(params[name]["c1"])
        ws += _prep_cbn(params[name]["c2"])
    ws += _prep_cbn(params["third"][0])
    ws += _prep_cbn(params["third"][1])
    ws += _prep_cbn(params["forth"][0])
    ws += _prep_cbn(params["forth"][1])
    ws += _pack_conv(params["final"])
    return ws


def _k5_call(pos2, ne2, ef2, m2, params):
    ws = _k5_weights(params)
    return pl.pallas_call(
        _k5_body,
        out_shape=jax.ShapeDtypeStruct((N, 3), F32),
    )(pos2, ne2, ef2, m2, *ws)


def kernel(position, message, charges, atom_types, edge_index, params):
    pos = position.astype(F32)
    xs, ys, zs = pos[:, 0], pos[:, 1], pos[:, 2]
    row = edge_index[0].astype(jnp.int32)
    col = edge_index[1].astype(jnp.int32)

    partials = _sc_agg(xs, ys, zs, row, col)
    part_t = partials.reshape(NW * LANES, N).T

    ch2 = charges.reshape(N, 1).astype(F32)
    at2 = atom_types.reshape(N, 5).astype(F32)
    ne2, f1 = _k2_call(part_t, ch2, at2, params["node"], params["edge"]["s_p"])

    rp = _k3_call(params["edge"]["relpos"])
    ef = _k4_call(f1.reshape(BS, NUM_MOL, FEAT), rp, params["edge"])
    ef2 = ef.reshape(N, 3)

    m2 = jnp.tile(message.transpose(0, 2, 1).reshape(N, WM), (1, 3))
    y = _k5_call(pos, ne2, ef2, m2, params)
    return y.reshape(BS, NUM_MOL, 3)[:, None]
